# baseline (speedup 1.0000x reference)
import jax
import jax.numpy as jnp
from jax import lax
from jax.experimental import pallas as pl
from jax.experimental.pallas import tpu as pltpu
from jax.experimental.pallas import tpu_sc as plsc

N_NODES = 10000
N_PAD = 10240
N_EDGES = 160000
AVG_NUM_NEIGHBORS = 16.0

NS = 16
NCHUNK = 4
CW = 32
EB = 40
EPT = N_EDGES // NS
NBLK = EPT // EB
NPT = N_PAD // NS
IG = 25
NGRP = NBLK // IG


def _a1_body(nf_ref, wv_ref, xt_ref):
    x = jnp.dot(nf_ref[...], wv_ref[...], preferred_element_type=jnp.float32)
    xt_ref[pl.ds(0, N_NODES), :] = x * (1.0 / jnp.sqrt(128.0))


def _stage_a1(node_feats, W_value):
    return pl.pallas_call(
        _a1_body,
        out_shape=jax.ShapeDtypeStruct((N_PAD, 128), jnp.float32),
    )(node_feats, W_value)


def _a2_body(ef_ref, ea_ref, w1_ref, w2_ref, w3_ref, w4_ref, s_ref, wz_ref):
    bf = jnp.bfloat16
    h = jnp.dot(ef_ref[...], w1_ref[...], preferred_element_type=jnp.float32)
    h = jax.nn.silu(h * (1.0 / jnp.sqrt(8.0)))
    h = jnp.dot(h.astype(bf), w2_ref[...].astype(bf),
                preferred_element_type=jnp.float32)
    h = jax.nn.silu(h * (1.0 / jnp.sqrt(64.0)))
    h = jnp.dot(h.astype(bf), w3_ref[...].astype(bf),
                preferred_element_type=jnp.float32)
    h = jax.nn.silu(h * (1.0 / jnp.sqrt(64.0)))
    tw = jnp.dot(h.astype(bf), w4_ref[...].astype(bf),
                 preferred_element_type=jnp.float32)
    tw = tw * (1.0 / jnp.sqrt(64.0))
    yf = jnp.dot(ea_ref[...], s_ref[...], preferred_element_type=jnp.float32)
    for c in range(NCHUNK):
        wz_ref[c] = tw[:, c * 128:(c + 1) * 128] * yf


def _stage_a2(edge_feats, edge_attrs, W_r1, W_r2, W_r3, W_r4):
    BE = 4000
    grid = (N_EDGES // BE,)
    w0 = W_r4[:, :128]
    w1 = W_r4[:, 128:]
    blocks = []
    for c in range(NCHUNK):
        w1c = w1[:, c * CW:(c + 1) * CW]
        blocks += [w0[:, c * CW:(c + 1) * CW], w1c, w1c, w1c]
    W_r4x = jnp.concatenate(blocks, axis=1)
    S = jnp.repeat(jnp.eye(4, dtype=jnp.float32), CW, axis=1)
    return pl.pallas_call(
        _a2_body,
        grid=grid,
        in_specs=[
            pl.BlockSpec((BE, 8), lambda i: (i, 0)),
            pl.BlockSpec((BE, 4), lambda i: (i, 0)),
            pl.BlockSpec((8, 64), lambda i: (0, 0)),
            pl.BlockSpec((64, 64), lambda i: (0, 0)),
            pl.BlockSpec((64, 64), lambda i: (0, 0)),
            pl.BlockSpec((64, 512), lambda i: (0, 0)),
            pl.BlockSpec((4, 128), lambda i: (0, 0)),
        ],
        out_specs=pl.BlockSpec((NCHUNK, BE, 128), lambda i: (0, i, 0)),
        out_shape=jax.ShapeDtypeStruct((NCHUNK, N_EDGES, 128), jnp.float32),
    )(edge_feats, edge_attrs, W_r1, W_r2, W_r3, W_r4x, S)


def _sc_body(x_hbm, wz_hbm, snd_hbm, rcv_hbm, acc_hbm,
             acc_sh, idx_s, idx_r, wz_a, wz_b, xs_a, xs_b, m_a, m_b,
             sem_wa, sem_wb, sem_xa, sem_xb, sem_ma, sem_mb):
    core = lax.axis_index("c")
    sub = lax.axis_index("s")
    wz_bufs = (wz_a, wz_b)
    xs_bufs = (xs_a, xs_b)
    m_bufs = (m_a, m_b)
    sem_w = (sem_wa, sem_wb)
    sem_x = (sem_xa, sem_xb)
    sem_m = (sem_ma, sem_mb)

    def issue_loads(cid, blk, par, gidx):
        ebase = cid * N_EDGES + sub * EPT + blk * EB
        pltpu.async_copy(wz_hbm.at[pl.ds(ebase, EB)], wz_bufs[par], sem_w[par])
        pltpu.async_copy(x_hbm.at[idx_s.at[gidx]], xs_bufs[par], sem_x[par])

    def wait_loads(par):
        pltpu.make_async_copy(wz_hbm.at[pl.ds(0, EB)], wz_bufs[par],
                              sem_w[par]).wait()
        pltpu.make_async_copy(x_hbm.at[idx_s.at[0]], xs_bufs[par],
                              sem_x[par]).wait()

    def wait_scatter(par):
        pltpu.make_async_copy(m_bufs[par], acc_sh.at[idx_r.at[0]],
                              sem_m[par]).wait()

    for k in range(2):
        cid = core * 2 + k
        ubase = cid * CW

        @pl.loop(0, EB)
        def _zero(i):
            for j in range(128 // 16):
                m_a[i, pl.ds(j * 16, 16)] = jnp.zeros((16,), jnp.float32)

        @pl.loop(0, NPT // EB)
        def _clear(i):
            pltpu.sync_copy(m_a, acc_sh.at[pl.ds(sub * NPT + i * EB, EB)])
        plsc.subcore_barrier()

        @pl.loop(0, NGRP)
        def _group(sg):
            pltpu.sync_copy(snd_hbm.at[sub, sg], idx_s)
            pltpu.sync_copy(rcv_hbm.at[sub, sg], idx_r)
            blk0 = sg * IG
            issue_loads(cid, blk0, 0, 0)
            for g in range(IG):
                par = g % 2
                if g + 1 < IG:
                    issue_loads(cid, blk0 + g + 1, 1 - par, g + 1)
                wait_loads(par)
                if g >= 2:
                    wait_scatter(par)
                xs_v = xs_bufs[par]
                wz_v = wz_bufs[par]
                m_v = m_bufs[par]

                @plsc.parallel_loop(0, EB)
                def _edge(e):
                    xs0 = xs_v[e, pl.ds(ubase, 16)]
                    xs1 = xs_v[e, pl.ds(ubase + 16, 16)]
                    for j in range(4):
                        m_v[e, pl.ds(j * 32, 16)] = (
                            wz_v[e, pl.ds(j * 32, 16)] * xs0)
                        m_v[e, pl.ds(j * 32 + 16, 16)] = (
                            wz_v[e, pl.ds(j * 32 + 16, 16)] * xs1)

                pltpu.async_copy(m_v, acc_sh.at[idx_r.at[g]], sem_m[par],
                                 add=True)
            wait_scatter(0)
            wait_scatter(1)

        plsc.subcore_barrier()
        pltpu.sync_copy(acc_sh.at[pl.ds(sub * NPT, NPT)],
                        acc_hbm.at[cid, pl.ds(sub * NPT, NPT)])
        plsc.subcore_barrier()


def _stage_sc(xfull, wz, snd3, rcv3):
    mesh = plsc.VectorSubcoreMesh(core_axis_name="c", subcore_axis_name="s")
    kern = pl.kernel(
        _sc_body,
        out_type=jax.ShapeDtypeStruct((NCHUNK, N_PAD, 128), jnp.float32),
        mesh=mesh,
        scratch_types=[
            pltpu.VMEM_SHARED((N_PAD, 128), jnp.float32),
            pltpu.VMEM((IG, EB), jnp.int32),
            pltpu.VMEM((IG, EB), jnp.int32),
            pltpu.VMEM((EB, 128), jnp.float32),
            pltpu.VMEM((EB, 128), jnp.float32),
            pltpu.VMEM((EB, 128), jnp.float32),
            pltpu.VMEM((EB, 128), jnp.float32),
            pltpu.VMEM((EB, 128), jnp.float32),
            pltpu.VMEM((EB, 128), jnp.float32),
            pltpu.SemaphoreType.DMA,
            pltpu.SemaphoreType.DMA,
            pltpu.SemaphoreType.DMA,
            pltpu.SemaphoreType.DMA,
            pltpu.SemaphoreType.DMA,
            pltpu.SemaphoreType.DMA,
        ],
    )
    return kern(xfull, wz.reshape(NCHUNK * N_EDGES, 128), snd3, rcv3)


def _b0_body(wl0_ref, ws0_ref, wl1_ref, ws1_ref, c0_ref, c1_ref):
    scale = 1.0 / (jnp.sqrt(128.0) * AVG_NUM_NEIGHBORS * jnp.sqrt(1280.0))
    wl0 = wl0_ref[...]
    wl1 = wl1_ref[...]
    for v in range(10):
        c0_ref[v] = jnp.dot(wl0, ws0_ref[:, v, :],
                            preferred_element_type=jnp.float32) * scale
        c1_ref[v] = jnp.dot(wl1, ws1_ref[:, v, :],
                            preferred_element_type=jnp.float32) * scale


def _stage_b0(W_lin0, W_skip0, W_lin1, W_skip1):
    return pl.pallas_call(
        _b0_body,
        out_shape=(jax.ShapeDtypeStruct((10, 128, 128), jnp.float32),
                   jax.ShapeDtypeStruct((10, 128, 128), jnp.float32)),
    )(W_lin0, W_skip0, W_lin1, W_skip1)


def _b1_body(acc_ref, attrs_ref, c0_ref, c1_ref, out_ref):
    a = acc_ref[...]
    attrs = attrs_ref[...]
    msgs = []
    for m in range(4):
        msgs.append(jnp.concatenate(
            [a[c, :, m * CW:(m + 1) * CW] for c in range(NCHUNK)], axis=1))
    bf = jnp.bfloat16
    outs = []
    for m in range(4):
        cp = (c0_ref if m == 0 else c1_ref)[...].astype(bf)
        mb = msgs[m].astype(bf)
        o = jnp.zeros_like(msgs[m])
        for v in range(10):
            o = o + attrs[:, v:v + 1] * jnp.dot(
                mb, cp[v], preferred_element_type=jnp.float32)
        outs.append(o)
    out_ref[...] = jnp.concatenate(outs, axis=1)


def _stage_b1(acc, node_attrs, Cp0, Cp1):
    BN = 1000
    grid = (N_NODES // BN,)
    return pl.pallas_call(
        _b1_body,
        grid=grid,
        in_specs=[
            pl.BlockSpec((NCHUNK, BN, 128), lambda i: (0, i, 0)),
            pl.BlockSpec((BN, 10), lambda i: (i, 0)),
            pl.BlockSpec((10, 128, 128), lambda i: (0, 0, 0)),
            pl.BlockSpec((10, 128, 128), lambda i: (0, 0, 0)),
        ],
        out_specs=pl.BlockSpec((BN, 512), lambda i: (i, 0)),
        out_shape=jax.ShapeDtypeStruct((N_NODES, 512), jnp.float32),
    )(acc, node_attrs, Cp0, Cp1)


def kernel(idx, node_attrs, node_feats, edge_attrs, edge_feats, edge_index,
           W_value, W_r1, W_r2, W_r3, W_r4, W_lin0, W_lin1, W_skip0, W_skip1):
    xfull = _stage_a1(node_feats, W_value)
    wz = _stage_a2(edge_feats, edge_attrs, W_r1, W_r2, W_r3, W_r4)
    snd3 = edge_index[0].reshape(NS, NGRP, IG, EB)
    rcv3 = edge_index[1].reshape(NS, NGRP, IG, EB)
    acc = _stage_sc(xfull, wz, snd3, rcv3)
    Cp0, Cp1 = _stage_b0(W_lin0, W_skip0, W_lin1, W_skip1)
    planar = _stage_b1(acc, node_attrs, Cp0, Cp1)
    out1 = jnp.stack(
        [planar[:, 128:256], planar[:, 256:384], planar[:, 384:512]],
        axis=-1).reshape(N_NODES, 384)
    return jnp.concatenate([planar[:, :128], out1], axis=-1)


# speedup vs baseline: 1.0811x; 1.0811x over previous
import jax
import jax.numpy as jnp
from jax import lax
from jax.experimental import pallas as pl
from jax.experimental.pallas import tpu as pltpu
from jax.experimental.pallas import tpu_sc as plsc

N_NODES = 10000
N_PAD = 10240
N_EDGES = 160000
AVG_NUM_NEIGHBORS = 16.0

NS = 16
NCHUNK = 4
CW = 32
EB = 40
EPT = N_EDGES // NS
NBLK = EPT // EB
NPT = N_PAD // NS
IG = 25
NGRP = NBLK // IG


def _a1_body(nf_ref, wv_ref, xt_ref):
    x = jnp.dot(nf_ref[...], wv_ref[...], preferred_element_type=jnp.float32)
    xt_ref[pl.ds(0, N_NODES), :] = x * (1.0 / jnp.sqrt(128.0))


def _stage_a1(node_feats, W_value):
    return pl.pallas_call(
        _a1_body,
        out_shape=jax.ShapeDtypeStruct((N_PAD, 128), jnp.float32),
    )(node_feats, W_value)


def _a2_body(ef_ref, ea_ref, w1_ref, w2_ref, w3_ref, w4_ref, s_ref, wz_ref):
    bf = jnp.bfloat16
    h = jnp.dot(ef_ref[...], w1_ref[...], preferred_element_type=jnp.float32)
    h = jax.nn.silu(h * (1.0 / jnp.sqrt(8.0)))
    h = jnp.dot(h.astype(bf), w2_ref[...].astype(bf),
                preferred_element_type=jnp.float32)
    h = jax.nn.silu(h * (1.0 / jnp.sqrt(64.0)))
    h = jnp.dot(h.astype(bf), w3_ref[...].astype(bf),
                preferred_element_type=jnp.float32)
    h = jax.nn.silu(h * (1.0 / jnp.sqrt(64.0)))
    tw = jnp.dot(h.astype(bf), w4_ref[...].astype(bf),
                 preferred_element_type=jnp.float32)
    tw = tw * (1.0 / jnp.sqrt(64.0))
    yf = jnp.dot(ea_ref[...], s_ref[...], preferred_element_type=jnp.float32)
    for c in range(NCHUNK):
        wz_ref[c] = tw[:, c * 128:(c + 1) * 128] * yf


def _stage_a2(edge_feats, edge_attrs, W_r1, W_r2, W_r3, W_r4):
    BE = 4000
    grid = (N_EDGES // BE,)
    w0 = W_r4[:, :128]
    w1 = W_r4[:, 128:]
    blocks = []
    for c in range(NCHUNK):
        w1c = w1[:, c * CW:(c + 1) * CW]
        blocks += [w0[:, c * CW:(c + 1) * CW], w1c, w1c, w1c]
    W_r4x = jnp.concatenate(blocks, axis=1)
    S = jnp.repeat(jnp.eye(4, dtype=jnp.float32), CW, axis=1)
    return pl.pallas_call(
        _a2_body,
        grid=grid,
        in_specs=[
            pl.BlockSpec((BE, 8), lambda i: (i, 0)),
            pl.BlockSpec((BE, 4), lambda i: (i, 0)),
            pl.BlockSpec((8, 64), lambda i: (0, 0)),
            pl.BlockSpec((64, 64), lambda i: (0, 0)),
            pl.BlockSpec((64, 64), lambda i: (0, 0)),
            pl.BlockSpec((64, 512), lambda i: (0, 0)),
            pl.BlockSpec((4, 128), lambda i: (0, 0)),
        ],
        out_specs=pl.BlockSpec((NCHUNK, BE, 128), lambda i: (0, i, 0)),
        out_shape=jax.ShapeDtypeStruct((NCHUNK, N_EDGES, 128), jnp.float32),
    )(edge_feats, edge_attrs, W_r1, W_r2, W_r3, W_r4x, S)


def _sc_body(x_hbm, wz_hbm, snd_hbm, rcv_hbm, acc_hbm,
             acc_sh, idx_s, idx_r, wz_a, wz_b, xs_a, xs_b, m_a, m_b,
             sem_wa, sem_wb, sem_xa, sem_xb, sem_ma, sem_mb):
    core = lax.axis_index("c")
    sub = lax.axis_index("s")
    wz_bufs = (wz_a, wz_b)
    xs_bufs = (xs_a, xs_b)
    m_bufs = (m_a, m_b)
    sem_w = (sem_wa, sem_wb)
    sem_x = (sem_xa, sem_xb)
    sem_m = (sem_ma, sem_mb)

    def issue_loads(cid, blk, par, gidx):
        ebase = cid * N_EDGES + sub * EPT + blk * EB
        pltpu.async_copy(wz_hbm.at[pl.ds(ebase, EB)], wz_bufs[par], sem_w[par])
        pltpu.async_copy(x_hbm.at[idx_s.at[gidx]], xs_bufs[par], sem_x[par])

    def wait_loads(par):
        pltpu.make_async_copy(wz_hbm.at[pl.ds(0, EB)], wz_bufs[par],
                              sem_w[par]).wait()
        pltpu.make_async_copy(x_hbm.at[idx_s.at[0]], xs_bufs[par],
                              sem_x[par]).wait()

    def wait_scatter(par):
        pltpu.make_async_copy(m_bufs[par], acc_sh.at[idx_r.at[0]],
                              sem_m[par]).wait()

    for k in range(2):
        cid = core * 2 + k
        ubase = cid * CW

        @pl.loop(0, EB)
        def _zero(i):
            for j in range(128 // 16):
                m_a[i, pl.ds(j * 16, 16)] = jnp.zeros((16,), jnp.float32)

        @pl.loop(0, NPT // EB)
        def _clear(i):
            pltpu.sync_copy(m_a, acc_sh.at[pl.ds(sub * NPT + i * EB, EB)])
        plsc.subcore_barrier()

        @pl.loop(0, NGRP)
        def _group(sg):
            pltpu.sync_copy(snd_hbm.at[sub, sg], idx_s)
            pltpu.sync_copy(rcv_hbm.at[sub, sg], idx_r)
            blk0 = sg * IG
            issue_loads(cid, blk0, 0, 0)
            for g in range(IG):
                par = g % 2
                if g + 1 < IG:
                    issue_loads(cid, blk0 + g + 1, 1 - par, g + 1)
                wait_loads(par)
                if g >= 2:
                    wait_scatter(par)
                xs_v = xs_bufs[par]
                wz_v = wz_bufs[par]
                m_v = m_bufs[par]

                @plsc.parallel_loop(0, EB)
                def _edge(e):
                    xs0 = xs_v[e, pl.ds(ubase, 16)]
                    xs1 = xs_v[e, pl.ds(ubase + 16, 16)]
                    for j in range(4):
                        m_v[e, pl.ds(j * 32, 16)] = (
                            wz_v[e, pl.ds(j * 32, 16)] * xs0)
                        m_v[e, pl.ds(j * 32 + 16, 16)] = (
                            wz_v[e, pl.ds(j * 32 + 16, 16)] * xs1)

                pltpu.async_copy(m_v, acc_sh.at[idx_r.at[g]], sem_m[par],
                                 add=True)
            wait_scatter(0)
            wait_scatter(1)

        plsc.subcore_barrier()
        pltpu.sync_copy(acc_sh.at[pl.ds(sub * NPT, NPT)],
                        acc_hbm.at[cid, pl.ds(sub * NPT, NPT)])
        plsc.subcore_barrier()


def _stage_sc(xfull, wz, snd3, rcv3):
    mesh = plsc.VectorSubcoreMesh(core_axis_name="c", subcore_axis_name="s",
                                  num_cores=2, num_subcores=NS)
    kern = pl.kernel(
        _sc_body,
        out_type=jax.ShapeDtypeStruct((NCHUNK, N_PAD, 128), jnp.float32),
        mesh=mesh,
        scratch_types=[
            pltpu.VMEM_SHARED((N_PAD, 128), jnp.float32),
            pltpu.VMEM((IG, EB), jnp.int32),
            pltpu.VMEM((IG, EB), jnp.int32),
            pltpu.VMEM((EB, 128), jnp.float32),
            pltpu.VMEM((EB, 128), jnp.float32),
            pltpu.VMEM((EB, 128), jnp.float32),
            pltpu.VMEM((EB, 128), jnp.float32),
            pltpu.VMEM((EB, 128), jnp.float32),
            pltpu.VMEM((EB, 128), jnp.float32),
            pltpu.SemaphoreType.DMA,
            pltpu.SemaphoreType.DMA,
            pltpu.SemaphoreType.DMA,
            pltpu.SemaphoreType.DMA,
            pltpu.SemaphoreType.DMA,
            pltpu.SemaphoreType.DMA,
        ],
    )
    return kern(xfull, wz.reshape(NCHUNK * N_EDGES, 128), snd3, rcv3)


def _b0_body(wl0_ref, ws0_ref, wl1_ref, ws1_ref, c0_ref, c1_ref):
    scale = 1.0 / (jnp.sqrt(128.0) * AVG_NUM_NEIGHBORS * jnp.sqrt(1280.0))
    wl0 = wl0_ref[...]
    wl1 = wl1_ref[...]
    for v in range(10):
        c0_ref[v] = jnp.dot(wl0, ws0_ref[:, v, :],
                            preferred_element_type=jnp.float32) * scale
        c1_ref[v] = jnp.dot(wl1, ws1_ref[:, v, :],
                            preferred_element_type=jnp.float32) * scale


def _stage_b0(W_lin0, W_skip0, W_lin1, W_skip1):
    return pl.pallas_call(
        _b0_body,
        out_shape=(jax.ShapeDtypeStruct((10, 128, 128), jnp.float32),
                   jax.ShapeDtypeStruct((10, 128, 128), jnp.float32)),
    )(W_lin0, W_skip0, W_lin1, W_skip1)


def _b1_body(acc_ref, attrs_ref, c0_ref, c1_ref, out_ref):
    a = acc_ref[...]
    attrs = attrs_ref[...]
    bf = jnp.bfloat16
    msgs = []
    for m in range(4):
        msgs.append(jnp.concatenate(
            [a[c, :, m * CW:(m + 1) * CW] for c in range(NCHUNK)],
            axis=1).astype(bf))
    m1cat = jnp.concatenate(msgs[1:], axis=1)
    o0 = jnp.zeros((attrs.shape[0], 128), jnp.float32)
    o1 = jnp.zeros((attrs.shape[0], 384), jnp.float32)
    for v in range(10):
        av = attrs[:, v:v + 1]
        o0 = o0 + av * jnp.dot(msgs[0], c0_ref[v],
                               preferred_element_type=jnp.float32)
        o1 = o1 + av * jnp.dot(m1cat, c1_ref[v],
                               preferred_element_type=jnp.float32)
    out_ref[...] = jnp.concatenate([o0, o1], axis=1)


def _stage_b1(acc, node_attrs, Cp0, CB1):
    BN = 1000
    grid = (N_NODES // BN,)
    return pl.pallas_call(
        _b1_body,
        grid=grid,
        in_specs=[
            pl.BlockSpec((NCHUNK, BN, 128), lambda i: (0, i, 0)),
            pl.BlockSpec((BN, 10), lambda i: (i, 0)),
            pl.BlockSpec((10, 128, 128), lambda i: (0, 0, 0)),
            pl.BlockSpec((10, 384, 384), lambda i: (0, 0, 0)),
        ],
        out_specs=pl.BlockSpec((BN, 512), lambda i: (i, 0)),
        out_shape=jax.ShapeDtypeStruct((N_NODES, 512), jnp.float32),
    )(acc, node_attrs, Cp0, CB1)


def kernel(idx, node_attrs, node_feats, edge_attrs, edge_feats, edge_index,
           W_value, W_r1, W_r2, W_r3, W_r4, W_lin0, W_lin1, W_skip0, W_skip1):
    xfull = _stage_a1(node_feats, W_value)
    wz = _stage_a2(edge_feats, edge_attrs, W_r1, W_r2, W_r3, W_r4)
    snd3 = edge_index[0].reshape(NS, NGRP, IG, EB)
    rcv3 = edge_index[1].reshape(NS, NGRP, IG, EB)
    acc = _stage_sc(xfull, wz, snd3, rcv3)
    Cp0, Cp1 = _stage_b0(W_lin0, W_skip0, W_lin1, W_skip1)
    CB1 = jnp.einsum('vtw,me->vmtwe', Cp1,
                     jnp.eye(3, dtype=jnp.float32)).reshape(10, 384, 384)
    return _stage_b1(acc, node_attrs, Cp0.astype(jnp.bfloat16),
                     CB1.astype(jnp.bfloat16))
